# baseline (speedup 1.0000x reference)
import functools

import jax
import jax.numpy as jnp
from jax import lax
from jax.experimental import pallas as pl
from jax.experimental.pallas import tpu as pltpu
from jax.experimental.pallas import tpu_sc as plsc

N = 10000
HID = 128
E = 320000
EMB_DIMS = (24, 10, 6, 4)

NP = 10112
XW = 128
ROW_BLK = 632
HEAD_BLK = 400

NC, NS, LANES = 2, 16, 16
NWORK = NC * NS
GRP = 128
KW = 80
EP = NWORK * KW * GRP
TILE_ROWS = NP // NS


def _stage1_body(xn_ref, code_ref, wn_ref, c_ref, b_ref, o_ref):
    xn = xn_ref[...]
    code = code_ref[...]
    lane = lax.broadcasted_iota(jnp.int32, (ROW_BLK, HID), 1)
    oh = jnp.zeros((ROW_BLK, HID), jnp.float32)
    for k in range(4):
        ck = (code >> (5 * k)) & 31
        oh += (lane == ck + 32 * k).astype(jnp.float32)
    acc = (jnp.dot(xn, wn_ref[...], preferred_element_type=jnp.float32)
           + jnp.dot(oh, c_ref[...], preferred_element_type=jnp.float32)
           + b_ref[...])
    o_ref[...] = jnp.maximum(acc, 0.0)


def _stage1(xn, code, wn, ctab, b):
    grid = NP // ROW_BLK
    return pl.pallas_call(
        _stage1_body,
        grid=(grid,),
        in_specs=[
            pl.BlockSpec((ROW_BLK, HID), lambda i: (i, 0)),
            pl.BlockSpec((ROW_BLK, 1), lambda i: (i, 0)),
            pl.BlockSpec((HID, HID), lambda i: (0, 0)),
            pl.BlockSpec((HID, HID), lambda i: (0, 0)),
            pl.BlockSpec((1, HID), lambda i: (0, 0)),
        ],
        out_specs=pl.BlockSpec((ROW_BLK, XW), lambda i: (i, 0)),
        out_shape=jax.ShapeDtypeStruct((NP, XW), jnp.float32),
    )(xn, code, wn, ctab, b)


SCAN_G = E // NC // GRP
SEL = 12288
ACCR = TILE_ROWS + 8
GRPB = 64


def _bucket_body(scan_hbm, sels_out, seld_out, cnt_out, buf_v, sels_v,
                 seld_v, cnt_v):
    c = lax.axis_index("c")
    s = lax.axis_index("s")
    base = s * TILE_ROWS
    lane = lax.iota(jnp.int32, LANES)
    pad_d = TILE_ROWS + (lane % 8)
    pad_s = jnp.zeros((LANES,), jnp.int32)

    def _fill(i, _):
        sels_v[pl.ds(i * LANES, LANES)] = pad_s
        seld_v[pl.ds(i * LANES, LANES)] = pad_d
        return ()

    lax.fori_loop(0, SEL // LANES, _fill, ())

    def _scan(g, off):
        pltpu.sync_copy(scan_hbm.at[c, g], buf_v)
        for j in range(GRP // LANES):
            s16 = buf_v[0, pl.ds(j * LANES, LANES)]
            d16 = buf_v[1, pl.ds(j * LANES, LANES)]
            m = (d16 >= base) & (d16 < base + TILE_ROWS)
            plsc.store_compressed(sels_v.at[pl.ds(off, LANES)], s16, mask=m)
            plsc.store_compressed(seld_v.at[pl.ds(off, LANES)],
                                  d16 - base, mask=m)
            off = off + jnp.sum(m.astype(jnp.int32))
        return off

    cnt = lax.fori_loop(0, SCAN_G, _scan, jnp.int32(0))
    cnt_v[...] = jnp.broadcast_to(cnt, (LANES,))
    pltpu.sync_copy(sels_v, sels_out.at[c, s])
    pltpu.sync_copy(seld_v, seld_out.at[c, s])
    pltpu.sync_copy(cnt_v, cnt_out.at[c, s])


def _agg2_body(x_hbm, sels_hbm, seld_hbm, cnt_hbm, agg_out, deg_out,
               src_v, dstl_v, rows_v, deg_v, acc, cnt_vv,
               semg0, semg1):
    c = lax.axis_index("c")
    s = lax.axis_index("s")
    base = s * TILE_ROWS

    pltpu.sync_copy(sels_hbm.at[c, s], src_v)
    pltpu.sync_copy(seld_hbm.at[c, s], dstl_v)
    pltpu.sync_copy(cnt_hbm.at[c, s], cnt_vv)
    cnt = jnp.max(cnt_vv[...])
    ngrp = lax.div(cnt + (GRPB - 1), GRPB)

    zv = jnp.zeros((LANES,), jnp.float32)

    def _zacc(i, _):
        for j in range(XW // LANES):
            acc[i, pl.ds(j * LANES, LANES)] = zv
        return ()

    lax.fori_loop(0, ACCR, _zacc, ())

    def _zdeg(i, _):
        deg_v[pl.ds(i * LANES, LANES)] = zv
        return ()

    lax.fori_loop(0, ACCR // LANES, _zdeg, ())

    sg = (semg0, semg1)
    ones = jnp.ones((LANES,), jnp.float32)

    def _stage(b, g, sem):
        pltpu.async_copy(x_hbm.at[src_v.at[pl.ds(g * GRPB, GRPB)]],
                         rows_v.at[b], sem)

    def _gwait(b, g, sem):
        pltpu.make_async_copy(x_hbm.at[src_v.at[pl.ds(g * GRPB, GRPB)]],
                              rows_v.at[b], sem).wait()

    for b in range(2):
        @pl.when(b < ngrp)
        def _(b=b):
            _stage(b, b, sg[b])

    cols = [lax.iota(jnp.int32, LANES) + j * LANES
            for j in range(XW // LANES)]
    _dnums = lax.GatherDimensionNumbers(
        offset_dims=(), collapsed_slice_dims=(0,), start_index_map=(0,))

    def _take16(vec, i):
        idx = jnp.full((LANES, 1), i, jnp.int32)
        return lax.gather(vec, idx, _dnums, (1,),
                          mode=lax.GatherScatterMode.PROMISE_IN_BOUNDS)

    def _accumulate(b, g):
        for q in range(GRPB // LANES):
            d16 = dstl_v[pl.ds(g * GRPB + q * LANES, LANES)]
            plsc.addupdate_scatter(deg_v, [d16], ones)
            for i in range(LANES):
                rvec = _take16(d16, i)
                for j in range(XW // LANES):
                    v = rows_v[b, q * LANES + i, pl.ds(j * LANES, LANES)]
                    plsc.addupdate_scatter(acc, [rvec, cols[j]], v)

    def _loop(t, _):
        g0 = 2 * t
        for b in range(2):
            @pl.when(g0 + b < ngrp)
            def _(b=b, g=g0 + b):
                _gwait(b, g, sg[b])
                _accumulate(b, g)

        for b in range(2):
            @pl.when(g0 + 2 + b < ngrp)
            def _(b=b, g=g0 + 2 + b):
                _stage(b, g, sg[b])
        return ()

    lax.fori_loop(0, (SEL // GRPB + 1) // 2, _loop, ())

    pltpu.sync_copy(acc.at[pl.ds(0, TILE_ROWS)],
                    agg_out.at[c, pl.ds(base, TILE_ROWS)])
    pltpu.sync_copy(deg_v.at[pl.ds(0, TILE_ROWS)],
                    deg_out.at[c, pl.ds(base, TILE_ROWS)])


@functools.lru_cache(maxsize=1)
def _get_sc_kernels():
    mesh = plsc.VectorSubcoreMesh(core_axis_name="c", subcore_axis_name="s",
                                  num_cores=NC, num_subcores=NS)
    params = pltpu.CompilerParams(use_tc_tiling_on_sc=False,
                                  needs_layout_passes=False)
    bucket = pl.kernel(
        _bucket_body,
        out_type=[jax.ShapeDtypeStruct((NC, NS, SEL), jnp.int32),
                  jax.ShapeDtypeStruct((NC, NS, SEL), jnp.int32),
                  jax.ShapeDtypeStruct((NC, NS, LANES), jnp.int32)],
        mesh=mesh,
        compiler_params=params,
        scratch_types=[
            pltpu.VMEM((2, GRP), jnp.int32),
            pltpu.VMEM((SEL,), jnp.int32),
            pltpu.VMEM((SEL,), jnp.int32),
            pltpu.VMEM((LANES,), jnp.int32),
        ],
    )
    agg = pl.kernel(
        _agg2_body,
        out_type=[jax.ShapeDtypeStruct((NC, NP, XW), jnp.float32),
                  jax.ShapeDtypeStruct((NC, NP), jnp.float32)],
        mesh=mesh,
        compiler_params=params,
        scratch_types=[
            pltpu.VMEM((SEL,), jnp.int32),
            pltpu.VMEM((SEL,), jnp.int32),
            pltpu.VMEM((2, GRPB, XW), jnp.float32),
            pltpu.VMEM((ACCR,), jnp.float32),
            pltpu.VMEM((ACCR, XW), jnp.float32),
            pltpu.VMEM((LANES,), jnp.int32),
            pltpu.SemaphoreType.DMA,
            pltpu.SemaphoreType.DMA,
        ],
    )
    return bucket, agg


def _sc_bucket(scan):
    return _get_sc_kernels()[0](scan)


def _sc_agg(x_ext, sels, seld, cnts):
    return _get_sc_kernels()[1](x_ext, sels, seld, cnts)


def _layer_core(aggp, degp, x, wl, bl, wr, nw, nb):
    deg = jnp.sum(degp, axis=1)
    inv = 1.0 / jnp.maximum(deg, 1.0)
    agg = (aggp[0] + aggp[1]) * inv[:, None]
    y = (jnp.dot(agg, wl, preferred_element_type=jnp.float32) + bl
         + jnp.dot(x, wr, preferred_element_type=jnp.float32))
    mu = jnp.mean(y, axis=-1, keepdims=True)
    var = jnp.mean((y - mu) ** 2, axis=-1, keepdims=True)
    h = (y - mu) * lax.rsqrt(var + 1e-5) * nw + nb
    return x + 0.5 * jnp.maximum(h, 0.0)


def _layer1_body(aggp_ref, degp_ref, xe_ref, wl_ref, bl_ref, wr_ref, nw_ref,
                 nb_ref, o_ref):
    o_ref[...] = _layer_core(aggp_ref[...], degp_ref[...], xe_ref[...],
                             wl_ref[...], bl_ref[...], wr_ref[...],
                             nw_ref[...], nb_ref[...])


def _layer2_body(aggp_ref, degp_ref, xe_ref, wl_ref, bl_ref, wr_ref, nw_ref,
                 nb_ref, h1_ref, h1b_ref, h2_ref, h2b_ref, o_ref):
    xnew = _layer_core(aggp_ref[...], degp_ref[...], xe_ref[...],
                       wl_ref[...], bl_ref[...], wr_ref[...], nw_ref[...],
                       nb_ref[...])
    h = jnp.maximum(
        jnp.dot(xnew, h1_ref[...], preferred_element_type=jnp.float32)
        + h1b_ref[...], 0.0)
    o_ref[...] = (jnp.dot(h, h2_ref[...], preferred_element_type=jnp.float32)
                  + h2b_ref[...])


def _wspec(r, c):
    return pl.BlockSpec((r, c), lambda i: (0, 0))


def _layer1(aggp, degp, xext, wl, bl, wr, nw, nb):
    return pl.pallas_call(
        _layer1_body,
        grid=(NP // ROW_BLK,),
        in_specs=[
            pl.BlockSpec((NC, ROW_BLK, XW), lambda i: (0, i, 0)),
            pl.BlockSpec((ROW_BLK, NC), lambda i: (i, 0)),
            pl.BlockSpec((ROW_BLK, XW), lambda i: (i, 0)),
            _wspec(HID, HID), _wspec(1, HID), _wspec(HID, HID),
            _wspec(1, HID), _wspec(1, HID),
        ],
        out_specs=pl.BlockSpec((ROW_BLK, XW), lambda i: (i, 0)),
        out_shape=jax.ShapeDtypeStruct((NP, XW), jnp.float32),
    )(aggp, degp, xext, wl, bl, wr, nw, nb)


def _layer2_head(aggp, degp, xext, wl, bl, wr, nw, nb, h1, h1b, h2, h2b):
    return pl.pallas_call(
        _layer2_body,
        grid=(N // HEAD_BLK,),
        in_specs=[
            pl.BlockSpec((NC, HEAD_BLK, XW), lambda i: (0, i, 0)),
            pl.BlockSpec((HEAD_BLK, NC), lambda i: (i, 0)),
            pl.BlockSpec((HEAD_BLK, XW), lambda i: (i, 0)),
            _wspec(HID, HID), _wspec(1, HID), _wspec(HID, HID),
            _wspec(1, HID), _wspec(1, HID),
            _wspec(HID, 64), _wspec(1, 64), _wspec(64, 1), _wspec(1, 1),
        ],
        out_specs=pl.BlockSpec((HEAD_BLK, 1), lambda i: (i, 0)),
        out_shape=jax.ShapeDtypeStruct((N, 1), jnp.float32),
    )(aggp, degp, xext, wl, bl, wr, nw, nb, h1, h1b, h2, h2b)


def kernel(x_num, x_cat, edge_index, emb0, emb1, emb2, emb3,
           lin0_w, lin0_b, c1_wl, c1_bl, c1_wr, c2_wl, c2_bl, c2_wr,
           n1_w, n1_b, n2_w, n2_b, h1_w, h1_b, h2_w, h2_b):
    f32 = jnp.float32

    xc = x_cat.astype(jnp.int32)
    code = (xc[:, 0] | (xc[:, 1] << 5) | (xc[:, 2] << 10)
            | (xc[:, 3] << 15)).reshape(N, 1)
    code = jnp.pad(code, ((0, NP - N), (0, 0)))

    offs, ctab = 0, jnp.zeros((HID, HID), f32)
    for k, (emb, d) in enumerate(zip((emb0, emb1, emb2, emb3), EMB_DIMS)):
        wk = lin0_w[:, HID + offs:HID + offs + d]
        ctab = lax.dynamic_update_slice(ctab, emb[:10] @ wk.T, (32 * k, 0))
        offs += d

    xn = jnp.pad(x_num, ((0, NP - N), (0, 0)))
    wn = lin0_w[:, :HID].T
    b0 = lin0_b.reshape(1, HID)

    src = edge_index[0].astype(jnp.int32).reshape(NC, SCAN_G, GRP)
    dst = edge_index[1].astype(jnp.int32).reshape(NC, SCAN_G, GRP)
    scan = jnp.stack([src, dst], axis=2)

    sels, seld, cnts = _sc_bucket(scan)

    x0 = _stage1(xn, code, wn, ctab, b0)

    agg1, degw = _sc_agg(x0, sels, seld, cnts)
    degp = degw.T
    x1 = _layer1(agg1, degp, x0, c1_wl.T, c1_bl.reshape(1, HID), c1_wr.T,
                 n1_w.reshape(1, HID), n1_b.reshape(1, HID))

    agg2, _ = _sc_agg(x1, sels, seld, cnts)
    out = _layer2_head(agg2, degp, x1, c2_wl.T, c2_bl.reshape(1, HID),
                       c2_wr.T, n2_w.reshape(1, HID), n2_b.reshape(1, HID),
                       h1_w.T, h1_b.reshape(1, 64), h2_w.T,
                       h2_b.reshape(1, 1))
    return out.reshape(N)


# speedup vs baseline: 1.8117x; 1.8117x over previous
import functools

import jax
import jax.numpy as jnp
from jax import lax
from jax.experimental import pallas as pl
from jax.experimental.pallas import tpu as pltpu
from jax.experimental.pallas import tpu_sc as plsc

N = 10000
HID = 128
E = 320000
EMB_DIMS = (24, 10, 6, 4)

NP = 10112
XW = 128
ROW_BLK = 632
HEAD_BLK = 400

NC, NS, LANES = 2, 16, 16
NWORK = NC * NS
GRP = 128
KW = 80
EP = NWORK * KW * GRP
TILE_ROWS = NP // NS


def _stage1_body(xn_ref, code_ref, wn_ref, c_ref, b_ref, o_ref):
    xn = xn_ref[...]
    code = code_ref[...]
    lane = lax.broadcasted_iota(jnp.int32, (ROW_BLK, HID), 1)
    oh = jnp.zeros((ROW_BLK, HID), jnp.float32)
    for k in range(4):
        ck = (code >> (5 * k)) & 31
        oh += (lane == ck + 32 * k).astype(jnp.float32)
    acc = (jnp.dot(xn, wn_ref[...], preferred_element_type=jnp.float32)
           + jnp.dot(oh, c_ref[...], preferred_element_type=jnp.float32)
           + b_ref[...])
    o_ref[...] = jnp.maximum(acc, 0.0)


def _stage1(xn, code, wn, ctab, b):
    grid = NP // ROW_BLK
    return pl.pallas_call(
        _stage1_body,
        grid=(grid,),
        in_specs=[
            pl.BlockSpec((ROW_BLK, HID), lambda i: (i, 0)),
            pl.BlockSpec((ROW_BLK, 1), lambda i: (i, 0)),
            pl.BlockSpec((HID, HID), lambda i: (0, 0)),
            pl.BlockSpec((HID, HID), lambda i: (0, 0)),
            pl.BlockSpec((1, HID), lambda i: (0, 0)),
        ],
        out_specs=pl.BlockSpec((ROW_BLK, XW), lambda i: (i, 0)),
        out_shape=jax.ShapeDtypeStruct((NP, XW), jnp.float32),
    )(xn, code, wn, ctab, b)


def _agg_body(x_hbm, edges_hbm, agg_out, deg_out, idx_v, rows_v, deg_v,
              agg_sh, semg0, semg1, sems0, sems1):
    c = lax.axis_index("c")
    s = lax.axis_index("s")
    wid = c * NS + s

    zv = jnp.zeros((LANES,), jnp.float32)

    def _zrow(i, _):
        for j in range(XW // LANES):
            rows_v[0, i, pl.ds(j * LANES, LANES)] = zv
        return ()

    lax.fori_loop(0, GRP, _zrow, ())

    def _zdeg(i, _):
        deg_v[pl.ds(i * LANES, LANES)] = zv
        return ()

    lax.fori_loop(0, NP // LANES, _zdeg, ())
    base = s * TILE_ROWS
    nfull = TILE_ROWS // GRP
    rem = TILE_ROWS % GRP
    for t in range(nfull):
        pltpu.sync_copy(rows_v.at[0], agg_sh.at[pl.ds(base + t * GRP, GRP)])
    pltpu.sync_copy(rows_v.at[0, pl.ds(0, rem)],
                    agg_sh.at[pl.ds(base + nfull * GRP, rem)])
    plsc.subcore_barrier()

    sg = (semg0, semg1)
    ss = (sems0, sems1)

    def _gather(b, sem):
        pltpu.async_copy(x_hbm.at[idx_v.at[b, 0]], rows_v.at[b], sem)

    def _gather_wait(b, sem):
        pltpu.make_async_copy(x_hbm.at[idx_v.at[b, 0]], rows_v.at[b],
                              sem).wait()

    def _scat(b, sem):
        pltpu.async_copy(rows_v.at[b], agg_sh.at[idx_v.at[b, 1]], sem,
                         add=True)

    def _scat_wait(b, sem):
        pltpu.make_async_copy(rows_v.at[b], agg_sh.at[idx_v.at[b, 1]],
                              sem).wait()

    for b in range(2):
        pltpu.sync_copy(edges_hbm.at[wid, b], idx_v.at[b])
        _gather(b, sg[b])

    ones = jnp.ones((LANES,), jnp.float32)

    def _deg_add(b):
        for j in range(GRP // LANES):
            idx16 = idx_v[b, 1, pl.ds(j * LANES, LANES)]
            plsc.addupdate_scatter(deg_v, [idx16], ones)

    def _loop(t, _):
        g0 = 2 * t
        for b in range(2):
            _gather_wait(b, sg[b])
            _scat(b, ss[b])
            _deg_add(b)
        for b in range(2):
            @pl.when(g0 + 2 + b < KW)
            def _(b=b, g=g0 + 2 + b):
                _scat_wait(b, ss[b])
                pltpu.sync_copy(edges_hbm.at[wid, g], idx_v.at[b])
                _gather(b, sg[b])
        return ()

    lax.fori_loop(0, KW // 2, _loop, ())
    for b in range(2):
        _scat_wait(b, ss[b])
    pltpu.sync_copy(deg_v, deg_out.at[wid])
    plsc.subcore_barrier()

    for t in range(nfull):
        pltpu.sync_copy(agg_sh.at[pl.ds(base + t * GRP, GRP)],
                        agg_out.at[c, pl.ds(base + t * GRP, GRP)])
    pltpu.sync_copy(agg_sh.at[pl.ds(base + nfull * GRP, rem)],
                    agg_out.at[c, pl.ds(base + nfull * GRP, rem)])


@functools.lru_cache(maxsize=1)
def _get_sc_agg():
    mesh = plsc.VectorSubcoreMesh(core_axis_name="c", subcore_axis_name="s",
                                  num_cores=NC, num_subcores=NS)
    return pl.kernel(
        _agg_body,
        out_type=[jax.ShapeDtypeStruct((NC, NP, XW), jnp.float32),
                  jax.ShapeDtypeStruct((NWORK, NP), jnp.float32)],
        mesh=mesh,
        compiler_params=pltpu.CompilerParams(use_tc_tiling_on_sc=False,
                                             needs_layout_passes=False),
        scratch_types=[
            pltpu.VMEM((2, 2, GRP), jnp.int32),
            pltpu.VMEM((2, GRP, XW), jnp.float32),
            pltpu.VMEM((NP,), jnp.float32),
            pltpu.VMEM_SHARED((NP, XW), jnp.float32),
            pltpu.SemaphoreType.DMA,
            pltpu.SemaphoreType.DMA,
            pltpu.SemaphoreType.DMA,
            pltpu.SemaphoreType.DMA,
        ],
    )


def _sc_agg(x_ext, edges):
    return _get_sc_agg()(x_ext, edges)


def _layer_core(aggp, degp, x, wl, bl, wr, nw, nb):
    deg = jnp.sum(degp, axis=1)
    inv = 1.0 / jnp.maximum(deg, 1.0)
    agg = (aggp[0] + aggp[1]) * inv[:, None]
    y = (jnp.dot(agg, wl, preferred_element_type=jnp.float32) + bl
         + jnp.dot(x, wr, preferred_element_type=jnp.float32))
    mu = jnp.mean(y, axis=-1, keepdims=True)
    var = jnp.mean((y - mu) ** 2, axis=-1, keepdims=True)
    h = (y - mu) * lax.rsqrt(var + 1e-5) * nw + nb
    return x + 0.5 * jnp.maximum(h, 0.0)


def _layer1_body(aggp_ref, degp_ref, xe_ref, wl_ref, bl_ref, wr_ref, nw_ref,
                 nb_ref, o_ref):
    o_ref[...] = _layer_core(aggp_ref[...], degp_ref[...], xe_ref[...],
                             wl_ref[...], bl_ref[...], wr_ref[...],
                             nw_ref[...], nb_ref[...])


def _layer2_body(aggp_ref, degp_ref, xe_ref, wl_ref, bl_ref, wr_ref, nw_ref,
                 nb_ref, h1_ref, h1b_ref, h2_ref, h2b_ref, o_ref):
    xnew = _layer_core(aggp_ref[...], degp_ref[...], xe_ref[...],
                       wl_ref[...], bl_ref[...], wr_ref[...], nw_ref[...],
                       nb_ref[...])
    h = jnp.maximum(
        jnp.dot(xnew, h1_ref[...], preferred_element_type=jnp.float32)
        + h1b_ref[...], 0.0)
    o_ref[...] = (jnp.dot(h, h2_ref[...], preferred_element_type=jnp.float32)
                  + h2b_ref[...])


def _wspec(r, c):
    return pl.BlockSpec((r, c), lambda i: (0, 0))


def _layer1(aggp, degp, xext, wl, bl, wr, nw, nb):
    return pl.pallas_call(
        _layer1_body,
        grid=(NP // ROW_BLK,),
        in_specs=[
            pl.BlockSpec((NC, ROW_BLK, XW), lambda i: (0, i, 0)),
            pl.BlockSpec((ROW_BLK, NWORK), lambda i: (i, 0)),
            pl.BlockSpec((ROW_BLK, XW), lambda i: (i, 0)),
            _wspec(HID, HID), _wspec(1, HID), _wspec(HID, HID),
            _wspec(1, HID), _wspec(1, HID),
        ],
        out_specs=pl.BlockSpec((ROW_BLK, XW), lambda i: (i, 0)),
        out_shape=jax.ShapeDtypeStruct((NP, XW), jnp.float32),
    )(aggp, degp, xext, wl, bl, wr, nw, nb)


def _layer2_head(aggp, degp, xext, wl, bl, wr, nw, nb, h1, h1b, h2, h2b):
    return pl.pallas_call(
        _layer2_body,
        grid=(N // HEAD_BLK,),
        in_specs=[
            pl.BlockSpec((NC, HEAD_BLK, XW), lambda i: (0, i, 0)),
            pl.BlockSpec((HEAD_BLK, NWORK), lambda i: (i, 0)),
            pl.BlockSpec((HEAD_BLK, XW), lambda i: (i, 0)),
            _wspec(HID, HID), _wspec(1, HID), _wspec(HID, HID),
            _wspec(1, HID), _wspec(1, HID),
            _wspec(HID, 64), _wspec(1, 64), _wspec(64, 1), _wspec(1, 1),
        ],
        out_specs=pl.BlockSpec((HEAD_BLK, 1), lambda i: (i, 0)),
        out_shape=jax.ShapeDtypeStruct((N, 1), jnp.float32),
    )(aggp, degp, xext, wl, bl, wr, nw, nb, h1, h1b, h2, h2b)


def kernel(x_num, x_cat, edge_index, emb0, emb1, emb2, emb3,
           lin0_w, lin0_b, c1_wl, c1_bl, c1_wr, c2_wl, c2_bl, c2_wr,
           n1_w, n1_b, n2_w, n2_b, h1_w, h1_b, h2_w, h2_b):
    f32 = jnp.float32

    xc = x_cat.astype(jnp.int32)
    code = (xc[:, 0] | (xc[:, 1] << 5) | (xc[:, 2] << 10)
            | (xc[:, 3] << 15)).reshape(N, 1)
    code = jnp.pad(code, ((0, NP - N), (0, 0)))

    offs, ctab = 0, jnp.zeros((HID, HID), f32)
    for k, (emb, d) in enumerate(zip((emb0, emb1, emb2, emb3), EMB_DIMS)):
        wk = lin0_w[:, HID + offs:HID + offs + d]
        ctab = lax.dynamic_update_slice(ctab, emb[:10] @ wk.T, (32 * k, 0))
        offs += d

    xn = jnp.pad(x_num, ((0, NP - N), (0, 0)))
    wn = lin0_w[:, :HID].T
    b0 = lin0_b.reshape(1, HID)

    per_w = E // NWORK
    pad_w = KW * GRP - per_w
    srcw = edge_index[0].astype(jnp.int32).reshape(NWORK, per_w)
    dstw = edge_index[1].astype(jnp.int32).reshape(NWORK, per_w)
    absorb = N + (jnp.arange(pad_w, dtype=jnp.int32) % (NP - N))
    src = jnp.concatenate(
        [srcw, jnp.zeros((NWORK, pad_w), jnp.int32)], axis=1)
    dst = jnp.concatenate(
        [dstw, jnp.broadcast_to(absorb, (NWORK, pad_w))], axis=1)
    edges = jnp.stack([src.reshape(NWORK, KW, GRP),
                       dst.reshape(NWORK, KW, GRP)], axis=2)

    x0 = _stage1(xn, code, wn, ctab, b0)

    agg1, degw = _sc_agg(x0, edges)
    degp = degw.T
    x1 = _layer1(agg1, degp, x0, c1_wl.T, c1_bl.reshape(1, HID), c1_wr.T,
                 n1_w.reshape(1, HID), n1_b.reshape(1, HID))

    agg2, _ = _sc_agg(x1, edges)
    out = _layer2_head(agg2, degp, x1, c2_wl.T, c2_bl.reshape(1, HID),
                       c2_wr.T, n2_w.reshape(1, HID), n2_b.reshape(1, HID),
                       h1_w.T, h1_b.reshape(1, 64), h2_w.T,
                       h2_b.reshape(1, 1))
    return out.reshape(N)


# speedup vs baseline: 3.0550x; 1.6863x over previous
import functools

import jax
import jax.numpy as jnp
from jax import lax
from jax.experimental import pallas as pl
from jax.experimental.pallas import tpu as pltpu
from jax.experimental.pallas import tpu_sc as plsc

N = 10000
HID = 128
E = 320000
EMB_DIMS = (24, 10, 6, 4)

NP = 10112
XW = 128
ROW_BLK = 632
HEAD_BLK = 400

NC, NS, LANES = 2, 16, 16
NWORK = NC * NS
NBUF = 3
GRP = 96
KW = 105
EP = NWORK * KW * GRP
TILE_ROWS = NP // NS


def _stage1_body(xn_ref, code_ref, wn_ref, c_ref, b_ref, o_ref):
    xn = xn_ref[...]
    code = code_ref[...]
    lane = lax.broadcasted_iota(jnp.int32, (ROW_BLK, HID), 1)
    oh = jnp.zeros((ROW_BLK, HID), jnp.float32)
    for k in range(4):
        ck = (code >> (5 * k)) & 31
        oh += (lane == ck + 32 * k).astype(jnp.float32)
    acc = (jnp.dot(xn, wn_ref[...], preferred_element_type=jnp.float32)
           + jnp.dot(oh, c_ref[...], preferred_element_type=jnp.float32)
           + b_ref[...])
    o_ref[...] = jnp.maximum(acc, 0.0)


def _stage1(xn, code, wn, ctab, b):
    grid = NP // ROW_BLK
    return pl.pallas_call(
        _stage1_body,
        grid=(grid,),
        in_specs=[
            pl.BlockSpec((ROW_BLK, HID), lambda i: (i, 0)),
            pl.BlockSpec((ROW_BLK, 1), lambda i: (i, 0)),
            pl.BlockSpec((HID, HID), lambda i: (0, 0)),
            pl.BlockSpec((HID, HID), lambda i: (0, 0)),
            pl.BlockSpec((1, HID), lambda i: (0, 0)),
        ],
        out_specs=pl.BlockSpec((ROW_BLK, XW), lambda i: (i, 0)),
        out_shape=jax.ShapeDtypeStruct((NP, XW), jnp.float32),
    )(xn, code, wn, ctab, b)


def _agg_body(x_hbm, edges_hbm, agg_out, deg_out, idx_v, rows_v, deg_v,
              agg_sh, semg0, semg1, semg2, sems0, sems1, sems2):
    c = lax.axis_index("c")
    s = lax.axis_index("s")
    wid = c * NS + s

    zv = jnp.zeros((LANES,), jnp.float32)

    def _zrow(i, _):
        for j in range(XW // LANES):
            rows_v[0, i, pl.ds(j * LANES, LANES)] = zv
        return ()

    lax.fori_loop(0, GRP, _zrow, ())

    def _zdeg(i, _):
        deg_v[pl.ds(i * LANES, LANES)] = zv
        return ()

    lax.fori_loop(0, NP // LANES, _zdeg, ())
    base = s * TILE_ROWS
    nfull = TILE_ROWS // GRP
    rem = TILE_ROWS % GRP
    for t in range(nfull):
        pltpu.sync_copy(rows_v.at[0], agg_sh.at[pl.ds(base + t * GRP, GRP)])
    pltpu.sync_copy(rows_v.at[0, pl.ds(0, rem)],
                    agg_sh.at[pl.ds(base + nfull * GRP, rem)])
    plsc.subcore_barrier()

    sg = (semg0, semg1, semg2)
    ss = (sems0, sems1, sems2)

    def _gather(b, sem):
        pltpu.async_copy(x_hbm.at[idx_v.at[b, 0]], rows_v.at[b], sem)

    def _gather_wait(b, sem):
        pltpu.make_async_copy(x_hbm.at[idx_v.at[b, 0]], rows_v.at[b],
                              sem).wait()

    def _scat(b, sem):
        pltpu.async_copy(rows_v.at[b], agg_sh.at[idx_v.at[b, 1]], sem,
                         add=True)

    def _scat_wait(b, sem):
        pltpu.make_async_copy(rows_v.at[b], agg_sh.at[idx_v.at[b, 1]],
                              sem).wait()

    for b in range(NBUF):
        pltpu.sync_copy(edges_hbm.at[wid, b], idx_v.at[b])
        _gather(b, sg[b])

    ones = jnp.ones((LANES,), jnp.float32)

    def _deg_add(b):
        for j in range(GRP // LANES):
            idx16 = idx_v[b, 1, pl.ds(j * LANES, LANES)]
            plsc.addupdate_scatter(deg_v, [idx16], ones)

    def _loop(t, _):
        g0 = NBUF * t
        for b in range(NBUF):
            _gather_wait(b, sg[b])
            _scat(b, ss[b])
            _deg_add(b)
        for b in range(NBUF):
            @pl.when(g0 + NBUF + b < KW)
            def _(b=b, g=g0 + NBUF + b):
                _scat_wait(b, ss[b])
                pltpu.sync_copy(edges_hbm.at[wid, g], idx_v.at[b])
                _gather(b, sg[b])
        return ()

    lax.fori_loop(0, KW // NBUF, _loop, ())
    for b in range(NBUF):
        _scat_wait(b, ss[b])
    pltpu.sync_copy(deg_v, deg_out.at[wid])
    plsc.subcore_barrier()

    for t in range(nfull):
        pltpu.sync_copy(agg_sh.at[pl.ds(base + t * GRP, GRP)],
                        agg_out.at[c, pl.ds(base + t * GRP, GRP)])
    pltpu.sync_copy(agg_sh.at[pl.ds(base + nfull * GRP, rem)],
                    agg_out.at[c, pl.ds(base + nfull * GRP, rem)])


@functools.lru_cache(maxsize=1)
def _get_sc_agg():
    mesh = plsc.VectorSubcoreMesh(core_axis_name="c", subcore_axis_name="s",
                                  num_cores=NC, num_subcores=NS)
    return pl.kernel(
        _agg_body,
        out_type=[jax.ShapeDtypeStruct((NC, NP, XW), jnp.float32),
                  jax.ShapeDtypeStruct((NWORK, NP), jnp.float32)],
        mesh=mesh,
        compiler_params=pltpu.CompilerParams(use_tc_tiling_on_sc=False,
                                             needs_layout_passes=False),
        scratch_types=[
            pltpu.VMEM((NBUF, 2, GRP), jnp.int32),
            pltpu.VMEM((NBUF, GRP, XW), jnp.float32),
            pltpu.VMEM((NP,), jnp.float32),
            pltpu.VMEM_SHARED((NP, XW), jnp.float32),
            pltpu.SemaphoreType.DMA,
            pltpu.SemaphoreType.DMA,
            pltpu.SemaphoreType.DMA,
            pltpu.SemaphoreType.DMA,
            pltpu.SemaphoreType.DMA,
            pltpu.SemaphoreType.DMA,
        ],
    )


def _sc_agg(x_ext, edges):
    return _get_sc_agg()(x_ext, edges)


def _layer_core(aggp, degp, x, wl, bl, wr, nw, nb):
    deg = jnp.sum(degp, axis=1)
    inv = 1.0 / jnp.maximum(deg, 1.0)
    agg = (aggp[0] + aggp[1]) * inv[:, None]
    y = (jnp.dot(agg, wl, preferred_element_type=jnp.float32) + bl
         + jnp.dot(x, wr, preferred_element_type=jnp.float32))
    mu = jnp.mean(y, axis=-1, keepdims=True)
    var = jnp.mean((y - mu) ** 2, axis=-1, keepdims=True)
    h = (y - mu) * lax.rsqrt(var + 1e-5) * nw + nb
    return x + 0.5 * jnp.maximum(h, 0.0)


def _layer1_body(aggp_ref, degp_ref, xe_ref, wl_ref, bl_ref, wr_ref, nw_ref,
                 nb_ref, o_ref):
    o_ref[...] = _layer_core(aggp_ref[...], degp_ref[...], xe_ref[...],
                             wl_ref[...], bl_ref[...], wr_ref[...],
                             nw_ref[...], nb_ref[...])


def _layer2_body(aggp_ref, degp_ref, xe_ref, wl_ref, bl_ref, wr_ref, nw_ref,
                 nb_ref, h1_ref, h1b_ref, h2_ref, h2b_ref, o_ref):
    xnew = _layer_core(aggp_ref[...], degp_ref[...], xe_ref[...],
                       wl_ref[...], bl_ref[...], wr_ref[...], nw_ref[...],
                       nb_ref[...])
    h = jnp.maximum(
        jnp.dot(xnew, h1_ref[...], preferred_element_type=jnp.float32)
        + h1b_ref[...], 0.0)
    o_ref[...] = (jnp.dot(h, h2_ref[...], preferred_element_type=jnp.float32)
                  + h2b_ref[...])


def _wspec(r, c):
    return pl.BlockSpec((r, c), lambda i: (0, 0))


def _layer1(aggp, degp, xext, wl, bl, wr, nw, nb):
    return pl.pallas_call(
        _layer1_body,
        grid=(NP // ROW_BLK,),
        in_specs=[
            pl.BlockSpec((NC, ROW_BLK, XW), lambda i: (0, i, 0)),
            pl.BlockSpec((ROW_BLK, NWORK), lambda i: (i, 0)),
            pl.BlockSpec((ROW_BLK, XW), lambda i: (i, 0)),
            _wspec(HID, HID), _wspec(1, HID), _wspec(HID, HID),
            _wspec(1, HID), _wspec(1, HID),
        ],
        out_specs=pl.BlockSpec((ROW_BLK, XW), lambda i: (i, 0)),
        out_shape=jax.ShapeDtypeStruct((NP, XW), jnp.float32),
    )(aggp, degp, xext, wl, bl, wr, nw, nb)


def _layer2_head(aggp, degp, xext, wl, bl, wr, nw, nb, h1, h1b, h2, h2b):
    return pl.pallas_call(
        _layer2_body,
        grid=(N // HEAD_BLK,),
        in_specs=[
            pl.BlockSpec((NC, HEAD_BLK, XW), lambda i: (0, i, 0)),
            pl.BlockSpec((HEAD_BLK, NWORK), lambda i: (i, 0)),
            pl.BlockSpec((HEAD_BLK, XW), lambda i: (i, 0)),
            _wspec(HID, HID), _wspec(1, HID), _wspec(HID, HID),
            _wspec(1, HID), _wspec(1, HID),
            _wspec(HID, 64), _wspec(1, 64), _wspec(64, 1), _wspec(1, 1),
        ],
        out_specs=pl.BlockSpec((HEAD_BLK, 1), lambda i: (i, 0)),
        out_shape=jax.ShapeDtypeStruct((N, 1), jnp.float32),
    )(aggp, degp, xext, wl, bl, wr, nw, nb, h1, h1b, h2, h2b)


def kernel(x_num, x_cat, edge_index, emb0, emb1, emb2, emb3,
           lin0_w, lin0_b, c1_wl, c1_bl, c1_wr, c2_wl, c2_bl, c2_wr,
           n1_w, n1_b, n2_w, n2_b, h1_w, h1_b, h2_w, h2_b):
    f32 = jnp.float32

    xc = x_cat.astype(jnp.int32)
    code = (xc[:, 0] | (xc[:, 1] << 5) | (xc[:, 2] << 10)
            | (xc[:, 3] << 15)).reshape(N, 1)
    code = jnp.pad(code, ((0, NP - N), (0, 0)))

    offs, ctab = 0, jnp.zeros((HID, HID), f32)
    for k, (emb, d) in enumerate(zip((emb0, emb1, emb2, emb3), EMB_DIMS)):
        wk = lin0_w[:, HID + offs:HID + offs + d]
        ctab = lax.dynamic_update_slice(ctab, emb[:10] @ wk.T, (32 * k, 0))
        offs += d

    xn = jnp.pad(x_num, ((0, NP - N), (0, 0)))
    wn = lin0_w[:, :HID].T
    b0 = lin0_b.reshape(1, HID)

    per_w = E // NWORK
    pad_w = KW * GRP - per_w
    srcw = edge_index[0].astype(jnp.int32).reshape(NWORK, per_w)
    dstw = edge_index[1].astype(jnp.int32).reshape(NWORK, per_w)
    absorb = N + (jnp.arange(pad_w, dtype=jnp.int32) % (NP - N))
    src = jnp.concatenate(
        [srcw, jnp.zeros((NWORK, pad_w), jnp.int32)], axis=1)
    dst = jnp.concatenate(
        [dstw, jnp.broadcast_to(absorb, (NWORK, pad_w))], axis=1)
    edges = jnp.stack([src.reshape(NWORK, KW, GRP),
                       dst.reshape(NWORK, KW, GRP)], axis=2)

    x0 = _stage1(xn, code, wn, ctab, b0)

    agg1, degw = _sc_agg(x0, edges)
    degp = degw.T
    x1 = _layer1(agg1, degp, x0, c1_wl.T, c1_bl.reshape(1, HID), c1_wr.T,
                 n1_w.reshape(1, HID), n1_b.reshape(1, HID))

    agg2, _ = _sc_agg(x1, edges)
    out = _layer2_head(agg2, degp, x1, c2_wl.T, c2_bl.reshape(1, HID),
                       c2_wr.T, n2_w.reshape(1, HID), n2_b.reshape(1, HID),
                       h1_w.T, h1_b.reshape(1, 64), h2_w.T,
                       h2_b.reshape(1, 1))
    return out.reshape(N)
